# baseline (speedup 1.0000x reference)
import jax
import jax.numpy as jnp
from jax.experimental import pallas as pl
from jax.experimental.pallas import tpu as pltpu

B, N, DMAX = 64, 2048, 64
D = 128

_W1 = 0
_W2 = 64
_WPB = 192
_CP = 208
_CB = 209
_C1 = 210
_C2 = 211
_G1 = 212
_BE1 = 213
_WROWS = 216


def _body(x_ref, t_ref, w_ref, o_ref):
    x = x_ref[0]
    t = t_ref[0, 0].reshape(N, 1)

    h = jnp.dot(x, w_ref[_W1:_W1 + DMAX, :],
                preferred_element_type=jnp.float32)
    h = h + w_ref[_C1, :]
    mu = jnp.mean(h, axis=-1, keepdims=True)
    var = jnp.mean((h - mu) ** 2, axis=-1, keepdims=True)
    h = (h - mu) * jax.lax.rsqrt(var + 1e-5) * w_ref[_G1, :] + w_ref[_BE1, :]
    h = jnp.maximum(h, 0.0)
    poly = jnp.dot(h, w_ref[_W2:_W2 + D, :],
                   preferred_element_type=jnp.float32)
    poly = poly + w_ref[_C2, :]

    x8 = x[:, :8]
    pt = jnp.dot(x8, w_ref[_WPB:_WPB + 8, :],
                 preferred_element_type=jnp.float32) + w_ref[_CP, :]
    bx = jnp.dot(x8, w_ref[_WPB + 8:_WPB + 16, :],
                 preferred_element_type=jnp.float32) + w_ref[_CB, :]

    o_ref[0] = jnp.where(t == 0, pt, jnp.where(t == 1, bx, poly))


def kernel(prompts, prompt_types, Wp, bp, Wb, bb, W1, b1, g1, be1, W2, b2,
           type_emb):
    t3 = prompt_types.reshape(B, 1, N)

    w = jnp.zeros((_WROWS, D), jnp.float32)
    w = w.at[_W1:_W1 + DMAX, :].set(W1)
    w = w.at[_W2:_W2 + D, :].set(W2)
    w = w.at[_WPB:_WPB + 2, :].set(Wp)
    w = w.at[_WPB + 8:_WPB + 12, :].set(Wb)
    w = w.at[_CP, :].set(bp + type_emb[0])
    w = w.at[_CB, :].set(bb + type_emb[1])
    w = w.at[_C1, :].set(b1)
    w = w.at[_C2, :].set(b2 + type_emb[2])
    w = w.at[_G1, :].set(g1)
    w = w.at[_BE1, :].set(be1)

    out = pl.pallas_call(
        _body,
        grid=(B,),
        in_specs=[
            pl.BlockSpec((1, N, DMAX), lambda i: (i, 0, 0)),
            pl.BlockSpec((1, 1, N), lambda i: (i, 0, 0)),
            pl.BlockSpec((_WROWS, D), lambda i: (0, 0)),
        ],
        out_specs=pl.BlockSpec((1, N, D), lambda i: (i, 0, 0)),
        out_shape=jax.ShapeDtypeStruct((B, N, D), jnp.float32),
        compiler_params=pltpu.CompilerParams(
            dimension_semantics=("arbitrary",),
        ),
    )(prompts, t3, w)
    return out
